# baseline (speedup 1.0000x reference)
import jax
import jax.numpy as jnp
from jax.experimental import pallas as pl
from jax.experimental.pallas import tpu as pltpu

EMBD = 128
B = 16384
R8 = B // 8
BLOCK_G = 512


def _depie_body(u_ref, t_ref, w_ref, b_ref, o_ref):
    pid = pl.program_id(0)
    t = t_ref[pl.ds(pid * BLOCK_G * 8, BLOCK_G * 8), :].reshape(BLOCK_G, 8, 1)
    coef = t * w_ref[...] + (b_ref[...] + 1.0)
    o_ref[...] = u_ref[...] * coef


@jax.jit
def _depie_tc(user3, timediffs, w3, b3):
    grid = (R8 // BLOCK_G,)
    return pl.pallas_call(
        _depie_body,
        grid=grid,
        in_specs=[
            pl.BlockSpec((BLOCK_G, 8, EMBD), lambda i: (i, 0, 0)),
            pl.BlockSpec((B, 1), lambda i: (0, 0)),
            pl.BlockSpec((1, 1, EMBD), lambda i: (0, 0, 0)),
            pl.BlockSpec((1, 1, EMBD), lambda i: (0, 0, 0)),
        ],
        out_specs=pl.BlockSpec((BLOCK_G, 8, EMBD), lambda i: (i, 0, 0)),
        out_shape=jax.ShapeDtypeStruct((R8, 8, EMBD), jnp.float32),
        compiler_params=pltpu.CompilerParams(
            dimension_semantics=("arbitrary",),
        ),
    )(user3, timediffs, w3, b3)


def kernel(user_embd, item_embd, timediffs, W_embd, b_embd):
    del item_embd
    user3 = user_embd.reshape(R8, 8, EMBD)
    w3 = W_embd.reshape(1, 1, EMBD)
    b3 = b_embd.reshape(1, 1, EMBD)
    out3 = _depie_tc(user3, timediffs, w3, b3)
    return out3.reshape(B, EMBD)
